# baseline (speedup 1.0000x reference)
import functools

import jax
import jax.numpy as jnp
from jax import lax
from jax.experimental import pallas as pl
from jax.experimental.pallas import tpu as pltpu
from jax.experimental.pallas import tpu_sc as plsc

N_ATOMS = 10000
N_EDGES = 320000
D = 128
N_RBF = 20

NC = 2
NS = 16
NW = NC * NS
HALVES = 2
E_H = N_EDGES // HALVES
E_W = E_H // NW
EC = 40
NCH = E_W // EC
SCH = 25
NSC = NCH // SCH
N_PAD = 10240
ROWS_T = N_PAD // NS


def _ssp(v):
    return jnp.maximum(v, 0.0) + jnp.log(1.0 + jnp.exp(-jnp.abs(v))) - 0.6931471805599453


def _pack_bf16_pairs(w):
    lo = jax.lax.bitcast_convert_type(w[:, : D // 2].astype(jnp.bfloat16), jnp.uint16)
    hi = jax.lax.bitcast_convert_type(w[:, D // 2 :].astype(jnp.bfloat16), jnp.uint16)
    return lo.astype(jnp.int32) | (hi.astype(jnp.int32) << 16)


def _h_body(x_ref, w_ref, b_ref, o_ref):
    o_ref[...] = (
        jnp.dot(x_ref[...], w_ref[...], preferred_element_type=jnp.float32) + b_ref[...]
    )


def _h_call(x, W_in, b_in2):
    return pl.pallas_call(
        _h_body,
        out_shape=jax.ShapeDtypeStruct((N_ATOMS, D), jnp.float32),
    )(x, W_in, b_in2)


WBLK = 3200


def _wij_body(ft_ref, w1_ref, b1_ref, w2_ref, b2_ref, o_ref):
    f_blk = ft_ref[...].T
    v = jnp.dot(f_blk, w1_ref[...], preferred_element_type=jnp.float32) + b1_ref[...]
    v = _ssp(v)
    w = (
        jnp.dot(
            v.astype(jnp.bfloat16),
            w2_ref[...].astype(jnp.bfloat16),
            preferred_element_type=jnp.float32,
        )
        + b2_ref[...]
    )
    o_ref[...] = _pack_bf16_pairs(w)


def _wij_call(f_t, Wf1, bf1_2, Wf2, bf2_2, half):
    grid = (E_H // WBLK,)
    off = half * (E_H // WBLK)
    return pl.pallas_call(
        _wij_body,
        grid=grid,
        in_specs=[
            pl.BlockSpec((N_RBF, WBLK), lambda i: (0, i + off)),
            pl.BlockSpec((N_RBF, D), lambda i: (0, 0)),
            pl.BlockSpec((1, D), lambda i: (0, 0)),
            pl.BlockSpec((D, D), lambda i: (0, 0)),
            pl.BlockSpec((1, D), lambda i: (0, 0)),
        ],
        out_specs=pl.BlockSpec((WBLK, D // 2), lambda i: (i, 0)),
        out_shape=jax.ShapeDtypeStruct((E_H, D // 2), jnp.int32),
    )(f_t, Wf1, bf1_2, Wf2, bf2_2)


_MASK_HI = -65536


def _sc_edge_body(
    h_hbm, wij_hbm, idxj_hbm, idxi_hbm, rcut_hbm, out_hbm,
    tj, ti, tr, rows0, rows1, wij0, wij1, res0, res1, agg_sh,
    sg0, sg1, sw0, sw1, ss0, ss1,
):
    c = lax.axis_index("c")
    s = lax.axis_index("s")
    wid = c * NS + s
    rows = (rows0, rows1)
    wjb = (wij0, wij1)
    res = (res0, res1)
    sg = (sg0, sg1)
    sw = (sw0, sw1)
    ss = (ss0, ss1)

    def zfill(i, _):
        res0[i // 8, pl.ds((i % 8) * 16, 16)] = jnp.zeros((16,), jnp.float32)
        return 0

    lax.fori_loop(0, EC * 8, zfill, 0)
    for k in range(ROWS_T // EC):
        pltpu.sync_copy(res0, agg_sh.at[pl.ds(s * ROWS_T + k * EC, EC)])
    plsc.subcore_barrier()

    wbase = wid * NCH

    def fire(sc, t, b):
        pltpu.async_copy(h_hbm.at[tj.at[t]], rows[b], sg[b])
        pltpu.async_copy(
            wij_hbm.at[pl.ds((wbase + sc * SCH + t) * EC, EC)], wjb[b], sw[b]
        )

    def wait_gw(t, b):
        pltpu.make_async_copy(h_hbm.at[tj.at[t]], rows[b], sg[b]).wait()
        pltpu.make_async_copy(wij_hbm.at[pl.ds(wbase * EC, EC)], wjb[b], sw[b]).wait()

    def drain_scatter(b):
        pltpu.make_async_copy(out_hbm.at[0, pl.ds(0, EC)], res[b], ss[b]).wait()

    def mul_row(b, e, rc):
        for g in range(4):
            w32 = wjb[b][e, pl.ds(g * 16, 16)]
            lo = jax.lax.bitcast_convert_type(w32 << 16, jnp.float32) * rc
            hi = jax.lax.bitcast_convert_type(w32 & _MASK_HI, jnp.float32) * rc
            sl_lo = pl.ds(g * 16, 16)
            sl_hi = pl.ds((g + 4) * 16, 16)
            res[b][e, sl_lo] = rows[b][e, sl_lo] * lo
            res[b][e, sl_hi] = rows[b][e, sl_hi] * hi

    def mul_16rows(t, b, e0, rcv):
        for r in range(16):
            mul_row(b, e0 + r, rcv[r])

    def superchunk(sc, _):
        pltpu.sync_copy(idxj_hbm.at[wid, sc], tj)
        pltpu.sync_copy(idxi_hbm.at[wid, sc], ti)
        pltpu.sync_copy(rcut_hbm.at[wid, sc], tr)
        fire(sc, 0, 0)
        fire(sc, 1, 1)

        def step(kk, _):
            for b in range(2):
                t = 2 * kk + b

                @pl.when(t < SCH)
                def _():
                    wait_gw(t, b)
                    mul_16rows(t, b, 0, tr[t, pl.ds(0, 16)])
                    mul_16rows(t, b, 16, tr[t, pl.ds(16, 16)])
                    rcv2 = tr[t, pl.ds(EC - 16, 16)]
                    for r in range(8, 16):
                        mul_row(b, EC - 16 + r, rcv2[r])
                    pltpu.async_copy(res[b], agg_sh.at[ti.at[t]], ss[b], add=True)
                    nxt = t + 2

                    @pl.when(nxt < SCH)
                    def _():
                        drain_scatter(b)
                        fire(sc, nxt, b)

            return 0

        lax.fori_loop(0, (SCH + 1) // 2, step, 0)
        drain_scatter(0)
        drain_scatter(1)
        return 0

    lax.fori_loop(0, NSC, superchunk, 0)
    plsc.subcore_barrier()
    pltpu.sync_copy(
        agg_sh.at[pl.ds(s * ROWS_T, ROWS_T)],
        out_hbm.at[c, pl.ds(s * ROWS_T, ROWS_T)],
    )


_sc_edge = functools.partial(
    pl.kernel,
    out_type=jax.ShapeDtypeStruct((NC, N_PAD, D), jnp.float32),
    mesh=plsc.VectorSubcoreMesh(core_axis_name="c", subcore_axis_name="s"),
    scratch_types=[
        pltpu.VMEM((SCH, EC), jnp.int32),
        pltpu.VMEM((SCH, EC), jnp.int32),
        pltpu.VMEM((SCH, EC), jnp.float32),
        pltpu.VMEM((EC, D), jnp.float32),
        pltpu.VMEM((EC, D), jnp.float32),
        pltpu.VMEM((EC, D // 2), jnp.int32),
        pltpu.VMEM((EC, D // 2), jnp.int32),
        pltpu.VMEM((EC, D), jnp.float32),
        pltpu.VMEM((EC, D), jnp.float32),
        pltpu.VMEM_SHARED((N_PAD, D), jnp.float32),
        pltpu.SemaphoreType.DMA,
        pltpu.SemaphoreType.DMA,
        pltpu.SemaphoreType.DMA,
        pltpu.SemaphoreType.DMA,
        pltpu.SemaphoreType.DMA,
        pltpu.SemaphoreType.DMA,
    ],
)(_sc_edge_body)


def _out_body(p_ref, q_ref, w1_ref, b1_ref, w2_ref, b2_ref, o_ref):
    agg = (
        p_ref[0, :N_ATOMS] + p_ref[1, :N_ATOMS]
        + q_ref[0, :N_ATOMS] + q_ref[1, :N_ATOMS]
    )
    v = _ssp(jnp.dot(agg, w1_ref[...], preferred_element_type=jnp.float32) + b1_ref[...])
    o_ref[...] = (
        jnp.dot(v, w2_ref[...], preferred_element_type=jnp.float32) + b2_ref[...]
    )


def _out_call(p1, p2, Wo1, bo1_2, Wo2, bo2_2):
    return pl.pallas_call(
        _out_body,
        out_shape=jax.ShapeDtypeStruct((N_ATOMS, D), jnp.float32),
    )(p1, p2, Wo1, bo1_2, Wo2, bo2_2)


def kernel(x, f_ij, idx_i, idx_j, rcut_ij, W_in, b_in, Wf1, bf1, Wf2, bf2, Wo1, bo1, Wo2, bo2):
    idx_i = idx_i.astype(jnp.int32).reshape(HALVES, NW, NSC, SCH, EC)
    idx_j = idx_j.astype(jnp.int32).reshape(HALVES, NW, NSC, SCH, EC)
    rcut = rcut_ij.reshape(HALVES, NW, NSC, SCH, EC)
    h = _h_call(x, W_in, b_in.reshape(1, D))
    f_t = f_ij.T
    bf1_2, bf2_2 = bf1.reshape(1, D), bf2.reshape(1, D)
    wij1 = _wij_call(f_t, Wf1, bf1_2, Wf2, bf2_2, 0)
    p1 = _sc_edge(h, wij1, idx_j[0], idx_i[0], rcut[0])
    wij2 = _wij_call(f_t, Wf1, bf1_2, Wf2, bf2_2, 1)
    p2 = _sc_edge(h, wij2, idx_j[1], idx_i[1], rcut[1])
    return _out_call(p1, p2, Wo1, bo1.reshape(1, D), Wo2, bo2.reshape(1, D))


# speedup vs baseline: 1.0436x; 1.0436x over previous
import functools

import jax
import jax.numpy as jnp
from jax import lax
from jax.experimental import pallas as pl
from jax.experimental.pallas import tpu as pltpu
from jax.experimental.pallas import tpu_sc as plsc

N_ATOMS = 10000
N_EDGES = 320000
D = 128
N_RBF = 20

NC = 2
NS = 16
NW = NC * NS
HALVES = 2
E_H = N_EDGES // HALVES
E_W = E_H // NW
EC = 40
NCH = E_W // EC
SCH = 25
NSC = NCH // SCH
N_PAD = 10240
ROWS_T = N_PAD // NS


def _ssp(v):
    return jnp.maximum(v, 0.0) + jnp.log(1.0 + jnp.exp(-jnp.abs(v))) - 0.6931471805599453


def _pack_bf16_pairs(w):
    lo = jax.lax.bitcast_convert_type(w[:, : D // 2].astype(jnp.bfloat16), jnp.uint16)
    hi = jax.lax.bitcast_convert_type(w[:, D // 2 :].astype(jnp.bfloat16), jnp.uint16)
    return lo.astype(jnp.int32) | (hi.astype(jnp.int32) << 16)


def _h_body(x_ref, w_ref, b_ref, o_ref):
    o_ref[...] = (
        jnp.dot(x_ref[...], w_ref[...], preferred_element_type=jnp.float32) + b_ref[...]
    )


def _h_call(x, W_in, b_in2):
    return pl.pallas_call(
        _h_body,
        out_shape=jax.ShapeDtypeStruct((N_ATOMS, D), jnp.float32),
    )(x, W_in, b_in2)


WBLK = 3200


def _wij_body(ft_ref, w1_ref, b1_ref, w2_ref, b2_ref, o_ref):
    f_blk = ft_ref[...].T
    v = jnp.dot(f_blk, w1_ref[...], preferred_element_type=jnp.float32) + b1_ref[...]
    v = _ssp(v)
    w = (
        jnp.dot(
            v.astype(jnp.bfloat16),
            w2_ref[...].astype(jnp.bfloat16),
            preferred_element_type=jnp.float32,
        )
        + b2_ref[...]
    )
    o_ref[...] = _pack_bf16_pairs(w)


def _wij_call(f_t, Wf1, bf1_2, Wf2, bf2_2, half):
    grid = (E_H // WBLK,)
    off = half * (E_H // WBLK)
    return pl.pallas_call(
        _wij_body,
        grid=grid,
        in_specs=[
            pl.BlockSpec((N_RBF, WBLK), lambda i: (0, i + off)),
            pl.BlockSpec((N_RBF, D), lambda i: (0, 0)),
            pl.BlockSpec((1, D), lambda i: (0, 0)),
            pl.BlockSpec((D, D), lambda i: (0, 0)),
            pl.BlockSpec((1, D), lambda i: (0, 0)),
        ],
        out_specs=pl.BlockSpec((WBLK, D // 2), lambda i: (i, 0)),
        out_shape=jax.ShapeDtypeStruct((E_H, D // 2), jnp.int32),
    )(f_t, Wf1, bf1_2, Wf2, bf2_2)


_MASK_HI = -65536


def _sc_edge_body(
    h_hbm, wij_hbm, idxj_hbm, idxi_hbm, rcut_hbm, out_hbm,
    tj, ti, tr, rows0, rows1, wij0, wij1, res0, res1, agg_sh,
    sg0, sg1, sw0, sw1, ss0, ss1,
):
    c = lax.axis_index("c")
    s = lax.axis_index("s")
    wid = c * NS + s
    rows = (rows0, rows1)
    wjb = (wij0, wij1)
    res = (res0, res1)
    sg = (sg0, sg1)
    sw = (sw0, sw1)
    ss = (ss0, ss1)

    def zfill(i, _):
        res0[i // 8, pl.ds((i % 8) * 16, 16)] = jnp.zeros((16,), jnp.float32)
        return 0

    lax.fori_loop(0, EC * 8, zfill, 0)
    for k in range(ROWS_T // EC):
        pltpu.sync_copy(res0, agg_sh.at[pl.ds(s * ROWS_T + k * EC, EC)])
    plsc.subcore_barrier()

    wbase = wid * NCH

    def fire(sc, t, b):
        pltpu.async_copy(h_hbm.at[tj.at[t]], rows[b], sg[b])
        pltpu.async_copy(
            wij_hbm.at[pl.ds((wbase + sc * SCH + t) * EC, EC)], wjb[b], sw[b]
        )

    def wait_gw(t, b):
        pltpu.make_async_copy(h_hbm.at[tj.at[t]], rows[b], sg[b]).wait()
        pltpu.make_async_copy(wij_hbm.at[pl.ds(wbase * EC, EC)], wjb[b], sw[b]).wait()

    def drain_scatter(b):
        pltpu.make_async_copy(out_hbm.at[0, pl.ds(0, EC)], res[b], ss[b]).wait()

    def mul_row(b, e, rc):
        for g in range(4):
            w32 = wjb[b][e, pl.ds(g * 16, 16)]
            lo = jax.lax.bitcast_convert_type(w32 << 16, jnp.float32) * rc
            hi = jax.lax.bitcast_convert_type(w32 & _MASK_HI, jnp.float32) * rc
            sl_lo = pl.ds(g * 16, 16)
            sl_hi = pl.ds((g + 4) * 16, 16)
            res[b][e, sl_lo] = rows[b][e, sl_lo] * lo
            res[b][e, sl_hi] = rows[b][e, sl_hi] * hi

    def mul_16rows(t, b, e0, rcv):
        for r in range(16):
            mul_row(b, e0 + r, rcv[r])

    def superchunk(sc, _):
        pltpu.sync_copy(idxj_hbm.at[wid, sc], tj)
        pltpu.sync_copy(idxi_hbm.at[wid, sc], ti)
        pltpu.sync_copy(rcut_hbm.at[wid, sc], tr)
        fire(sc, 0, 0)
        fire(sc, 1, 1)

        def step(kk, _):
            for b in range(2):
                t = 2 * kk + b

                @pl.when(t < SCH)
                def _():
                    wait_gw(t, b)

                    @pl.when(t >= 2)
                    def _():
                        drain_scatter(b)

                    mul_16rows(t, b, 0, tr[t, pl.ds(0, 16)])
                    mul_16rows(t, b, 16, tr[t, pl.ds(16, 16)])
                    rcv2 = tr[t, pl.ds(EC - 16, 16)]
                    for r in range(8, 16):
                        mul_row(b, EC - 16 + r, rcv2[r])
                    pltpu.async_copy(res[b], agg_sh.at[ti.at[t]], ss[b], add=True)
                    nxt = t + 2

                    @pl.when(nxt < SCH)
                    def _():
                        fire(sc, nxt, b)

            return 0

        lax.fori_loop(0, (SCH + 1) // 2, step, 0)
        drain_scatter(0)
        drain_scatter(1)
        return 0

    lax.fori_loop(0, NSC, superchunk, 0)
    plsc.subcore_barrier()
    pltpu.sync_copy(
        agg_sh.at[pl.ds(s * ROWS_T, ROWS_T)],
        out_hbm.at[c, pl.ds(s * ROWS_T, ROWS_T)],
    )


_sc_edge = functools.partial(
    pl.kernel,
    out_type=jax.ShapeDtypeStruct((NC, N_PAD, D), jnp.float32),
    mesh=plsc.VectorSubcoreMesh(core_axis_name="c", subcore_axis_name="s"),
    scratch_types=[
        pltpu.VMEM((SCH, EC), jnp.int32),
        pltpu.VMEM((SCH, EC), jnp.int32),
        pltpu.VMEM((SCH, EC), jnp.float32),
        pltpu.VMEM((EC, D), jnp.float32),
        pltpu.VMEM((EC, D), jnp.float32),
        pltpu.VMEM((EC, D // 2), jnp.int32),
        pltpu.VMEM((EC, D // 2), jnp.int32),
        pltpu.VMEM((EC, D), jnp.float32),
        pltpu.VMEM((EC, D), jnp.float32),
        pltpu.VMEM_SHARED((N_PAD, D), jnp.float32),
        pltpu.SemaphoreType.DMA,
        pltpu.SemaphoreType.DMA,
        pltpu.SemaphoreType.DMA,
        pltpu.SemaphoreType.DMA,
        pltpu.SemaphoreType.DMA,
        pltpu.SemaphoreType.DMA,
    ],
)(_sc_edge_body)


def _out_body(p_ref, q_ref, w1_ref, b1_ref, w2_ref, b2_ref, o_ref):
    agg = (
        p_ref[0, :N_ATOMS] + p_ref[1, :N_ATOMS]
        + q_ref[0, :N_ATOMS] + q_ref[1, :N_ATOMS]
    )
    v = _ssp(jnp.dot(agg, w1_ref[...], preferred_element_type=jnp.float32) + b1_ref[...])
    o_ref[...] = (
        jnp.dot(v, w2_ref[...], preferred_element_type=jnp.float32) + b2_ref[...]
    )


def _out_call(p1, p2, Wo1, bo1_2, Wo2, bo2_2):
    return pl.pallas_call(
        _out_body,
        out_shape=jax.ShapeDtypeStruct((N_ATOMS, D), jnp.float32),
    )(p1, p2, Wo1, bo1_2, Wo2, bo2_2)


def kernel(x, f_ij, idx_i, idx_j, rcut_ij, W_in, b_in, Wf1, bf1, Wf2, bf2, Wo1, bo1, Wo2, bo2):
    idx_i = idx_i.astype(jnp.int32).reshape(HALVES, NW, NSC, SCH, EC)
    idx_j = idx_j.astype(jnp.int32).reshape(HALVES, NW, NSC, SCH, EC)
    rcut = rcut_ij.reshape(HALVES, NW, NSC, SCH, EC)
    h = _h_call(x, W_in, b_in.reshape(1, D))
    f_t = f_ij.T
    bf1_2, bf2_2 = bf1.reshape(1, D), bf2.reshape(1, D)
    wij1 = _wij_call(f_t, Wf1, bf1_2, Wf2, bf2_2, 0)
    p1 = _sc_edge(h, wij1, idx_j[0], idx_i[0], rcut[0])
    wij2 = _wij_call(f_t, Wf1, bf1_2, Wf2, bf2_2, 1)
    p2 = _sc_edge(h, wij2, idx_j[1], idx_i[1], rcut[1])
    return _out_call(p1, p2, Wo1, bo1.reshape(1, D), Wo2, bo2.reshape(1, D))
